# baseline (speedup 1.0000x reference)
import functools

import jax
import jax.numpy as jnp
from jax import lax
from jax.experimental import pallas as pl
from jax.experimental.pallas import tpu as pltpu
from jax.experimental.pallas import tpu_sc as plsc

_NC = 2
_NS = 16
_L = 16
_NW = _NC * _NS

_STRIDE = 8
_RADIU = 12.0
_LIMIT_LO = -1.0
_LIMIT_HI = 64.0
_BIG = 99999999.0


@functools.lru_cache(maxsize=None)
def _build(B, H, W, M):
  HW = H * W
  WPB = _NW // B
  PPW = HW // WPB
  CH = PPW // _L
  mesh = plsc.VectorSubcoreMesh(core_axis_name="c", subcore_axis_name="s",
                                num_cores=_NC, num_subcores=_NS)

  @functools.partial(
      pl.kernel,
      out_type=(
          jax.ShapeDtypeStruct((B * HW,), jnp.int32),
          jax.ShapeDtypeStruct((B * HW,), jnp.float32),
          jax.ShapeDtypeStruct((B * HW * 4,), jnp.float32),
      ),
      mesh=mesh,
      compiler_params=pltpu.CompilerParams(needs_layout_passes=False),
      scratch_types=[
          pltpu.VMEM((M, 4), jnp.float32),
          pltpu.VMEM((B, M), jnp.int32),
          pltpu.VMEM((PPW * 4,), jnp.float32),
          pltpu.VMEM((PPW,), jnp.int32),
          pltpu.VMEM((PPW,), jnp.float32),
          pltpu.VMEM((PPW,), jnp.float32),
      ],
  )
  def sc_kernel(gt_hbm, cls_hbm, clsout_hbm, cntout_hbm, regout_hbm,
                boxes_v, classes_v, rint, clsp, cntp, areap):
    wid = lax.axis_index("s") * _NC + lax.axis_index("c")
    b = wid // WPB
    q = wid % WPB

    pltpu.sync_copy(gt_hbm.at[b], boxes_v)
    pltpu.sync_copy(cls_hbm, classes_v)

    neg1 = jnp.full((_L,), -1.0, jnp.float32)
    zero_i = jnp.zeros((_L,), jnp.int32)
    big = jnp.full((_L,), _BIG, jnp.float32)

    def init_body(i, carry):
      base4 = i * (4 * _L)
      rint[pl.ds(base4, _L)] = neg1
      rint[pl.ds(base4 + _L, _L)] = neg1
      rint[pl.ds(base4 + 2 * _L, _L)] = neg1
      rint[pl.ds(base4 + 3 * _L, _L)] = neg1
      sl = pl.ds(i * _L, _L)
      clsp[sl] = zero_i
      cntp[sl] = neg1
      areap[sl] = big
      return carry

    lax.fori_loop(0, CH, init_body, 0)

    lane = lax.iota(jnp.int32, _L)
    dxl = lane % 3
    dyl = lane // 3
    lane_ok = lane < 9
    p_base = q * PPW
    col0 = jnp.zeros((_L,), jnp.int32)
    col1 = jnp.full((_L,), 1, jnp.int32)
    col2 = jnp.full((_L,), 2, jnp.int32)
    col3 = jnp.full((_L,), 3, jnp.int32)
    bvec = lax.broadcast(b, (_L,))

    def box_body(m, carry):
      mvec = lax.broadcast(m, (_L,))
      x0 = plsc.load_gather(boxes_v, [mvec, col0])
      y0 = plsc.load_gather(boxes_v, [mvec, col1])
      x1 = plsc.load_gather(boxes_v, [mvec, col2])
      y1 = plsc.load_gather(boxes_v, [mvec, col3])
      cm = plsc.load_gather(classes_v, [bvec, mvec])
      cx = (x0 + x1) * 0.5
      cy = (y0 + y1) * 0.5
      area = (x1 - x0) * (y1 - y0)
      i0 = ((cx + (1024.0 - 16.0)) * 0.125).astype(jnp.int32) - 127
      j0 = ((cy + (1024.0 - 16.0)) * 0.125).astype(jnp.int32) - 127
      ii = i0 + dxl
      jj = j0 + dyl
      valid = (lane_ok & (ii >= 0) & (ii < W) & (jj >= 0) & (jj < H))
      p_local = jj * W + ii - p_base
      in_r = (p_local >= 0) & (p_local < PPW)
      pc = jnp.clip(p_local, 0, PPW - 1)
      xv = (ii * _STRIDE + _STRIDE // 2).astype(jnp.float32)
      yv = (jj * _STRIDE + _STRIDE // 2).astype(jnp.float32)
      l = xv - x0
      t = yv - y0
      r = x1 - xv
      bb = y1 - yv
      off_min = jnp.minimum(jnp.minimum(l, t), jnp.minimum(r, bb))
      off_max = jnp.maximum(jnp.maximum(l, t), jnp.maximum(r, bb))
      c_off = jnp.maximum(jnp.abs(xv - cx), jnp.abs(yv - cy))
      mask = (valid & in_r & (off_min > 0.0)
              & (off_max > _LIMIT_LO) & (off_max <= _LIMIT_HI)
              & (c_off < _RADIU))
      best = plsc.load_gather(areap, [pc])
      upd = mask & (area < best)
      ratio = (jnp.minimum(l, r) * jnp.minimum(t, bb)) / (
          jnp.maximum(l, r) * jnp.maximum(t, bb) + 1e-10)
      xi = plsc.bitcast(ratio, jnp.int32)
      y = plsc.bitcast(0x5F3759DF - (xi >> 1), jnp.float32)
      y = y * (1.5 - 0.5 * ratio * y * y)
      y = y * (1.5 - 0.5 * ratio * y * y)
      y = y * (1.5 - 0.5 * ratio * y * y)
      cnt = ratio * y
      pc4 = pc * 4
      plsc.store_scatter(areap, [pc], area, mask=upd)
      plsc.store_scatter(cntp, [pc], cnt, mask=upd)
      plsc.store_scatter(clsp, [pc], cm, mask=upd)
      plsc.store_scatter(rint, [pc4], l, mask=upd)
      plsc.store_scatter(rint, [pc4 + 1], t, mask=upd)
      plsc.store_scatter(rint, [pc4 + 2], r, mask=upd)
      plsc.store_scatter(rint, [pc4 + 3], bb, mask=upd)
      return carry

    lax.fori_loop(0, M, box_body, 0)

    pltpu.sync_copy(clsp, clsout_hbm.at[pl.ds(wid * PPW, PPW)])
    pltpu.sync_copy(cntp, cntout_hbm.at[pl.ds(wid * PPW, PPW)])
    pltpu.sync_copy(rint, regout_hbm.at[pl.ds(wid * PPW * 4, PPW * 4)])

  return sc_kernel


@jax.jit
def kernel(cls_logits, cnt_logits, reg_preds, gt_boxes, classes):
  B, _, H, W = cls_logits.shape
  M = classes.shape[1]
  HW = H * W
  sc_kernel = _build(B, H, W, M)
  cls_flat, cnt_flat, reg_flat = sc_kernel(gt_boxes.astype(jnp.float32),
                                           classes.astype(jnp.int32))
  cls_t = cls_flat.reshape(B, HW, 1)
  cnt_t = cnt_flat.reshape(B, HW, 1)
  reg_t = reg_flat.reshape(B, HW, 4)
  return cls_t, cnt_t, reg_t


# speedup vs baseline: 4.3413x; 4.3413x over previous
import functools

import jax
import jax.numpy as jnp
from jax import lax
from jax.experimental import pallas as pl
from jax.experimental.pallas import tpu as pltpu
from jax.experimental.pallas import tpu_sc as plsc

_NC = 2
_NS = 16
_L = 16
_NW = _NC * _NS

_STRIDE = 8
_RADIU = 12.0
_LIMIT_LO = -1.0
_LIMIT_HI = 64.0
_BIG = 99999999.0


@functools.lru_cache(maxsize=None)
def _build(B, H, W, M):
  HW = H * W
  WPB = _NW // B
  PPW = HW // WPB
  CH = PPW // _L
  mesh = plsc.VectorSubcoreMesh(core_axis_name="c", subcore_axis_name="s",
                                num_cores=_NC, num_subcores=_NS)

  @functools.partial(
      pl.kernel,
      out_type=(
          jax.ShapeDtypeStruct((B * HW,), jnp.int32),
          jax.ShapeDtypeStruct((B * HW,), jnp.float32),
          jax.ShapeDtypeStruct((B * 4 * HW,), jnp.float32),
      ),
      mesh=mesh,
      compiler_params=pltpu.CompilerParams(needs_layout_passes=False),
      scratch_types=[
          pltpu.VMEM((M, 4), jnp.float32),
          pltpu.VMEM((B, M), jnp.int32),
          pltpu.VMEM((PPW,), jnp.float32),
          pltpu.VMEM((PPW,), jnp.float32),
          pltpu.VMEM((PPW,), jnp.float32),
          pltpu.VMEM((PPW,), jnp.float32),
          pltpu.VMEM((PPW,), jnp.int32),
          pltpu.VMEM((PPW,), jnp.float32),
          pltpu.VMEM((PPW,), jnp.float32),
      ],
  )
  def sc_kernel(gt_hbm, cls_hbm, clsout_hbm, cntout_hbm, regout_hbm,
                boxes_v, classes_v, rl, rt, rr, rb, clsp, cntp, areap):
    wid = lax.axis_index("s") * _NC + lax.axis_index("c")
    b = wid // WPB
    q = wid % WPB

    pltpu.sync_copy(gt_hbm.at[b], boxes_v)
    pltpu.sync_copy(cls_hbm, classes_v)

    neg1 = jnp.full((_L,), -1.0, jnp.float32)
    zero_i = jnp.zeros((_L,), jnp.int32)
    big = jnp.full((_L,), _BIG, jnp.float32)

    def init_body(i, carry):
      sl = pl.ds(i * _L, _L)
      rl[sl] = neg1
      rt[sl] = neg1
      rr[sl] = neg1
      rb[sl] = neg1
      clsp[sl] = zero_i
      cntp[sl] = neg1
      areap[sl] = big
      return carry

    lax.fori_loop(0, CH, init_body, 0)

    lane = lax.iota(jnp.int32, _L)
    dxl = lane % 3
    dyl = lane // 3
    lane_ok = lane < 9
    p_base = q * PPW
    col0 = jnp.zeros((_L,), jnp.int32)
    col1 = jnp.full((_L,), 1, jnp.int32)
    col2 = jnp.full((_L,), 2, jnp.int32)
    col3 = jnp.full((_L,), 3, jnp.int32)
    bvec = lax.broadcast(b, (_L,))

    def box_body(m, carry):
      mvec = lax.broadcast(m, (_L,))
      x0 = plsc.load_gather(boxes_v, [mvec, col0])
      y0 = plsc.load_gather(boxes_v, [mvec, col1])
      x1 = plsc.load_gather(boxes_v, [mvec, col2])
      y1 = plsc.load_gather(boxes_v, [mvec, col3])
      cm = plsc.load_gather(classes_v, [bvec, mvec])
      cx = (x0 + x1) * 0.5
      cy = (y0 + y1) * 0.5
      area = (x1 - x0) * (y1 - y0)
      i0 = ((cx + (1024.0 - 16.0)) * 0.125).astype(jnp.int32) - 127
      j0 = ((cy + (1024.0 - 16.0)) * 0.125).astype(jnp.int32) - 127
      ii = i0 + dxl
      jj = j0 + dyl
      valid = (lane_ok & (ii >= 0) & (ii < W) & (jj >= 0) & (jj < H))
      p_local = jj * W + ii - p_base
      in_r = (p_local >= 0) & (p_local < PPW)
      pc = jnp.clip(p_local, 0, PPW - 1)
      xv = (ii * _STRIDE + _STRIDE // 2).astype(jnp.float32)
      yv = (jj * _STRIDE + _STRIDE // 2).astype(jnp.float32)
      l = xv - x0
      t = yv - y0
      r = x1 - xv
      bb = y1 - yv
      off_min = jnp.minimum(jnp.minimum(l, t), jnp.minimum(r, bb))
      off_max = jnp.maximum(jnp.maximum(l, t), jnp.maximum(r, bb))
      c_off = jnp.maximum(jnp.abs(xv - cx), jnp.abs(yv - cy))
      mask = (valid & in_r & (off_min > 0.0)
              & (off_max > _LIMIT_LO) & (off_max <= _LIMIT_HI)
              & (c_off < _RADIU))
      best = plsc.load_gather(areap, [pc])
      upd = mask & (area < best)
      ratio = (jnp.minimum(l, r) * jnp.minimum(t, bb)) / (
          jnp.maximum(l, r) * jnp.maximum(t, bb) + 1e-10)
      xi = plsc.bitcast(ratio, jnp.int32)
      y = plsc.bitcast(0x5F3759DF - (xi >> 1), jnp.float32)
      y = y * (1.5 - 0.5 * ratio * y * y)
      y = y * (1.5 - 0.5 * ratio * y * y)
      y = y * (1.5 - 0.5 * ratio * y * y)
      cnt = ratio * y
      plsc.store_scatter(areap, [pc], area, mask=upd)
      plsc.store_scatter(cntp, [pc], cnt, mask=upd)
      plsc.store_scatter(clsp, [pc], cm, mask=upd)
      plsc.store_scatter(rl, [pc], l, mask=upd)
      plsc.store_scatter(rt, [pc], t, mask=upd)
      plsc.store_scatter(rr, [pc], r, mask=upd)
      plsc.store_scatter(rb, [pc], bb, mask=upd)
      return carry

    lax.fori_loop(0, M, box_body, 0)

    pltpu.sync_copy(clsp, clsout_hbm.at[pl.ds(wid * PPW, PPW)])
    pltpu.sync_copy(cntp, cntout_hbm.at[pl.ds(wid * PPW, PPW)])
    pltpu.sync_copy(rl, regout_hbm.at[pl.ds((b * 4 + 0) * HW + q * PPW, PPW)])
    pltpu.sync_copy(rt, regout_hbm.at[pl.ds((b * 4 + 1) * HW + q * PPW, PPW)])
    pltpu.sync_copy(rr, regout_hbm.at[pl.ds((b * 4 + 2) * HW + q * PPW, PPW)])
    pltpu.sync_copy(rb, regout_hbm.at[pl.ds((b * 4 + 3) * HW + q * PPW, PPW)])

  return sc_kernel


@jax.jit
def kernel(cls_logits, cnt_logits, reg_preds, gt_boxes, classes):
  B, _, H, W = cls_logits.shape
  M = classes.shape[1]
  HW = H * W
  sc_kernel = _build(B, H, W, M)
  cls_flat, cnt_flat, reg_flat = sc_kernel(gt_boxes.astype(jnp.float32),
                                           classes.astype(jnp.int32))
  cls_t = cls_flat.reshape(B, HW, 1)
  cnt_t = cnt_flat.reshape(B, HW, 1)
  reg_t = jnp.transpose(reg_flat.reshape(B, 4, HW), (0, 2, 1))
  return cls_t, cnt_t, reg_t


# speedup vs baseline: 4.5651x; 1.0516x over previous
import functools

import jax
import jax.numpy as jnp
from jax import lax
from jax.experimental import pallas as pl
from jax.experimental.pallas import tpu as pltpu
from jax.experimental.pallas import tpu_sc as plsc

_NC = 2
_NS = 16
_L = 16
_NW = _NC * _NS

_STRIDE = 8
_RADIU = 12.0
_LIMIT_LO = -1.0
_LIMIT_HI = 64.0
_BIG = 99999999.0


@functools.lru_cache(maxsize=None)
def _build(B, H, W, M):
  HW = H * W
  WPB = _NW // B
  PPW = HW // WPB
  CH = PPW // _L
  mesh = plsc.VectorSubcoreMesh(core_axis_name="c", subcore_axis_name="s",
                                num_cores=_NC, num_subcores=_NS)

  @functools.partial(
      pl.kernel,
      out_type=(
          jax.ShapeDtypeStruct((B * HW,), jnp.int32),
          jax.ShapeDtypeStruct((B * HW,), jnp.float32),
          jax.ShapeDtypeStruct((B * 4 * HW,), jnp.float32),
      ),
      mesh=mesh,
      compiler_params=pltpu.CompilerParams(needs_layout_passes=False),
      scratch_types=[
          pltpu.VMEM((M, 4), jnp.float32),
          pltpu.VMEM((B, M), jnp.int32),
          pltpu.VMEM((PPW,), jnp.float32),
          pltpu.VMEM((PPW,), jnp.float32),
          pltpu.VMEM((PPW,), jnp.float32),
          pltpu.VMEM((PPW,), jnp.float32),
          pltpu.VMEM((PPW,), jnp.int32),
          pltpu.VMEM((PPW,), jnp.float32),
          pltpu.VMEM((PPW,), jnp.float32),
          pltpu.SemaphoreType.DMA,
      ],
  )
  def sc_kernel(gt_hbm, cls_hbm, clsout_hbm, cntout_hbm, regout_hbm,
                boxes_v, classes_v, rl, rt, rr, rb, clsp, cntp, areap, sem):
    wid = lax.axis_index("s") * _NC + lax.axis_index("c")
    b = wid // WPB
    q = wid % WPB

    in1 = pltpu.async_copy(gt_hbm.at[b], boxes_v, sem)
    in2 = pltpu.async_copy(cls_hbm, classes_v, sem)

    neg1 = jnp.full((_L,), -1.0, jnp.float32)
    zero_i = jnp.zeros((_L,), jnp.int32)
    big = jnp.full((_L,), _BIG, jnp.float32)

    def init_body(i, carry):
      for u in range(4):
        sl = pl.ds((i * 4 + u) * _L, _L)
        rl[sl] = neg1
        rt[sl] = neg1
        rr[sl] = neg1
        rb[sl] = neg1
        clsp[sl] = zero_i
        cntp[sl] = neg1
        areap[sl] = big
      return carry

    lax.fori_loop(0, CH // 4, init_body, 0)
    in1.wait()
    in2.wait()

    lane = lax.iota(jnp.int32, _L)
    dxl = lane % 3
    dyl = lane // 3
    lane_ok = lane < 9
    p_base = q * PPW
    col0 = jnp.zeros((_L,), jnp.int32)
    col1 = jnp.full((_L,), 1, jnp.int32)
    col2 = jnp.full((_L,), 2, jnp.int32)
    col3 = jnp.full((_L,), 3, jnp.int32)
    bvec = lax.broadcast(b, (_L,))

    def box_body(m, carry):
      mvec = lax.broadcast(m, (_L,))
      x0 = plsc.load_gather(boxes_v, [mvec, col0])
      y0 = plsc.load_gather(boxes_v, [mvec, col1])
      x1 = plsc.load_gather(boxes_v, [mvec, col2])
      y1 = plsc.load_gather(boxes_v, [mvec, col3])
      cm = plsc.load_gather(classes_v, [bvec, mvec])
      cx = (x0 + x1) * 0.5
      cy = (y0 + y1) * 0.5
      area = (x1 - x0) * (y1 - y0)
      i0 = ((cx + (1024.0 - 16.0)) * 0.125).astype(jnp.int32) - 127
      j0 = ((cy + (1024.0 - 16.0)) * 0.125).astype(jnp.int32) - 127
      ii = i0 + dxl
      jj = j0 + dyl
      valid = (lane_ok & (ii >= 0) & (ii < W) & (jj >= 0) & (jj < H))
      p_local = jj * W + ii - p_base
      in_r = (p_local >= 0) & (p_local < PPW)
      pc = jnp.clip(p_local, 0, PPW - 1)
      xv = (ii * _STRIDE + _STRIDE // 2).astype(jnp.float32)
      yv = (jj * _STRIDE + _STRIDE // 2).astype(jnp.float32)
      l = xv - x0
      t = yv - y0
      r = x1 - xv
      bb = y1 - yv
      off_min = jnp.minimum(jnp.minimum(l, t), jnp.minimum(r, bb))
      off_max = jnp.maximum(jnp.maximum(l, t), jnp.maximum(r, bb))
      c_off = jnp.maximum(jnp.abs(xv - cx), jnp.abs(yv - cy))
      mask = (valid & in_r & (off_min > 0.0)
              & (off_max > _LIMIT_LO) & (off_max <= _LIMIT_HI)
              & (c_off < _RADIU))
      best = plsc.load_gather(areap, [pc])
      upd = mask & (area < best)
      ratio = (jnp.minimum(l, r) * jnp.minimum(t, bb)) / (
          jnp.maximum(l, r) * jnp.maximum(t, bb) + 1e-10)
      xi = plsc.bitcast(ratio, jnp.int32)
      y = plsc.bitcast(0x5F3759DF - (xi >> 1), jnp.float32)
      y = y * (1.5 - 0.5 * ratio * y * y)
      y = y * (1.5 - 0.5 * ratio * y * y)
      y = y * (1.5 - 0.5 * ratio * y * y)
      cnt = ratio * y
      plsc.store_scatter(areap, [pc], area, mask=upd)
      plsc.store_scatter(cntp, [pc], cnt, mask=upd)
      plsc.store_scatter(clsp, [pc], cm, mask=upd)
      plsc.store_scatter(rl, [pc], l, mask=upd)
      plsc.store_scatter(rt, [pc], t, mask=upd)
      plsc.store_scatter(rr, [pc], r, mask=upd)
      plsc.store_scatter(rb, [pc], bb, mask=upd)
      return carry

    lax.fori_loop(0, M, box_body, 0)

    outs = [
        pltpu.async_copy(clsp, clsout_hbm.at[pl.ds(wid * PPW, PPW)], sem),
        pltpu.async_copy(cntp, cntout_hbm.at[pl.ds(wid * PPW, PPW)], sem),
        pltpu.async_copy(
            rl, regout_hbm.at[pl.ds((b * 4 + 0) * HW + q * PPW, PPW)], sem),
        pltpu.async_copy(
            rt, regout_hbm.at[pl.ds((b * 4 + 1) * HW + q * PPW, PPW)], sem),
        pltpu.async_copy(
            rr, regout_hbm.at[pl.ds((b * 4 + 2) * HW + q * PPW, PPW)], sem),
        pltpu.async_copy(
            rb, regout_hbm.at[pl.ds((b * 4 + 3) * HW + q * PPW, PPW)], sem),
    ]
    for h in outs:
      h.wait()

  return sc_kernel


@jax.jit
def kernel(cls_logits, cnt_logits, reg_preds, gt_boxes, classes):
  B, _, H, W = cls_logits.shape
  M = classes.shape[1]
  HW = H * W
  sc_kernel = _build(B, H, W, M)
  cls_flat, cnt_flat, reg_flat = sc_kernel(gt_boxes.astype(jnp.float32),
                                           classes.astype(jnp.int32))
  cls_t = cls_flat.reshape(B, HW, 1)
  cnt_t = cnt_flat.reshape(B, HW, 1)
  reg_t = jnp.transpose(reg_flat.reshape(B, 4, HW), (0, 2, 1))
  return cls_t, cnt_t, reg_t


# speedup vs baseline: 4.5753x; 1.0022x over previous
import functools

import jax
import jax.numpy as jnp
from jax import lax
from jax.experimental import pallas as pl
from jax.experimental.pallas import tpu as pltpu
from jax.experimental.pallas import tpu_sc as plsc

_NC = 2
_NS = 16
_L = 16
_NW = _NC * _NS

_STRIDE = 8
_RADIU = 12.0
_LIMIT_LO = -1.0
_LIMIT_HI = 64.0
_BIG = 99999999.0


@functools.lru_cache(maxsize=None)
def _build(B, H, W, M):
  HW = H * W
  WPB = _NW // B
  PPW = HW // WPB
  CH = PPW // _L
  mesh = plsc.VectorSubcoreMesh(core_axis_name="c", subcore_axis_name="s",
                                num_cores=_NC, num_subcores=_NS)

  @functools.partial(
      pl.kernel,
      out_type=(
          jax.ShapeDtypeStruct((B * HW,), jnp.int32),
          jax.ShapeDtypeStruct((B * HW,), jnp.float32),
          jax.ShapeDtypeStruct((B * 4 * HW,), jnp.float32),
      ),
      mesh=mesh,
      compiler_params=pltpu.CompilerParams(needs_layout_passes=False),
      scratch_types=[
          pltpu.VMEM((M, 4), jnp.float32),
          pltpu.VMEM((B, M), jnp.int32),
          pltpu.VMEM((PPW,), jnp.float32),
          pltpu.VMEM((PPW,), jnp.float32),
          pltpu.VMEM((PPW,), jnp.float32),
          pltpu.VMEM((PPW,), jnp.float32),
          pltpu.VMEM((PPW,), jnp.int32),
          pltpu.VMEM((PPW,), jnp.float32),
          pltpu.VMEM((PPW,), jnp.float32),
          pltpu.SemaphoreType.DMA,
      ],
  )
  def sc_kernel(gt_hbm, cls_hbm, clsout_hbm, cntout_hbm, regout_hbm,
                boxes_v, classes_v, rl, rt, rr, rb, clsp, cntp, areap, sem):
    wid = lax.axis_index("s") * _NC + lax.axis_index("c")
    b = wid // WPB
    q = wid % WPB

    in1 = pltpu.async_copy(gt_hbm.at[b], boxes_v, sem)
    in2 = pltpu.async_copy(cls_hbm, classes_v, sem)

    neg1 = jnp.full((_L,), -1.0, jnp.float32)
    zero_i = jnp.zeros((_L,), jnp.int32)
    big = jnp.full((_L,), _BIG, jnp.float32)

    def init_body(i, carry):
      for u in range(4):
        sl = pl.ds((i * 4 + u) * _L, _L)
        rl[sl] = neg1
        rt[sl] = neg1
        rr[sl] = neg1
        rb[sl] = neg1
        clsp[sl] = zero_i
        cntp[sl] = neg1
        areap[sl] = big
      return carry

    lax.fori_loop(0, CH // 4, init_body, 0)
    in1.wait()
    in2.wait()

    lane = lax.iota(jnp.int32, _L)
    dxl = lane % 3
    dyl = lane // 3
    lane_ok = lane < 9
    p_base = q * PPW
    col0 = jnp.zeros((_L,), jnp.int32)
    col1 = jnp.full((_L,), 1, jnp.int32)
    col2 = jnp.full((_L,), 2, jnp.int32)
    col3 = jnp.full((_L,), 3, jnp.int32)
    bvec = lax.broadcast(b, (_L,))

    ROWS = PPW // W
    row0 = q * ROWS

    def box_body(m, carry):
      mvec = lax.broadcast(m, (_L,))
      y0 = plsc.load_gather(boxes_v, [mvec, col1])
      y1 = plsc.load_gather(boxes_v, [mvec, col3])
      cy = (y0 + y1) * 0.5
      j0 = ((cy + (1024.0 - 16.0)) * 0.125).astype(jnp.int32) - 127
      j0s = j0[0]

      @pl.when((j0s + 2 >= row0) & (j0s < row0 + ROWS))
      def _process():
        x0 = plsc.load_gather(boxes_v, [mvec, col0])
        x1 = plsc.load_gather(boxes_v, [mvec, col2])
        cm = plsc.load_gather(classes_v, [bvec, mvec])
        cx = (x0 + x1) * 0.5
        area = (x1 - x0) * (y1 - y0)
        i0 = ((cx + (1024.0 - 16.0)) * 0.125).astype(jnp.int32) - 127
        ii = i0 + dxl
        jj = j0 + dyl
        valid = (lane_ok & (ii >= 0) & (ii < W) & (jj >= 0) & (jj < H))
        p_local = jj * W + ii - p_base
        in_r = (p_local >= 0) & (p_local < PPW)
        pc = p_local & (PPW - 1)
        xv = (ii * _STRIDE + _STRIDE // 2).astype(jnp.float32)
        yv = (jj * _STRIDE + _STRIDE // 2).astype(jnp.float32)
        l = xv - x0
        t = yv - y0
        r = x1 - xv
        bb = y1 - yv
        off_min = jnp.minimum(jnp.minimum(l, t), jnp.minimum(r, bb))
        off_max = jnp.maximum(jnp.maximum(l, t), jnp.maximum(r, bb))
        c_off = jnp.maximum(jnp.abs(xv - cx), jnp.abs(yv - cy))
        mask = (valid & in_r & (off_min > 0.0)
                & (off_max > _LIMIT_LO) & (off_max <= _LIMIT_HI)
                & (c_off < _RADIU))
        best = plsc.load_gather(areap, [pc])
        upd = mask & (area < best)
        ratio = (jnp.minimum(l, r) * jnp.minimum(t, bb)) / (
            jnp.maximum(l, r) * jnp.maximum(t, bb) + 1e-10)
        xi = plsc.bitcast(ratio, jnp.int32)
        y = plsc.bitcast(0x5F3759DF - (xi >> 1), jnp.float32)
        y = y * (1.5 - 0.5 * ratio * y * y)
        y = y * (1.5 - 0.5 * ratio * y * y)
        y = y * (1.5 - 0.5 * ratio * y * y)
        cnt = ratio * y
        plsc.store_scatter(areap, [pc], area, mask=upd)
        plsc.store_scatter(cntp, [pc], cnt, mask=upd)
        plsc.store_scatter(clsp, [pc], cm, mask=upd)
        plsc.store_scatter(rl, [pc], l, mask=upd)
        plsc.store_scatter(rt, [pc], t, mask=upd)
        plsc.store_scatter(rr, [pc], r, mask=upd)
        plsc.store_scatter(rb, [pc], bb, mask=upd)

      return carry

    lax.fori_loop(0, M, box_body, 0)

    outs = [
        pltpu.async_copy(clsp, clsout_hbm.at[pl.ds(wid * PPW, PPW)], sem),
        pltpu.async_copy(cntp, cntout_hbm.at[pl.ds(wid * PPW, PPW)], sem),
        pltpu.async_copy(
            rl, regout_hbm.at[pl.ds((b * 4 + 0) * HW + q * PPW, PPW)], sem),
        pltpu.async_copy(
            rt, regout_hbm.at[pl.ds((b * 4 + 1) * HW + q * PPW, PPW)], sem),
        pltpu.async_copy(
            rr, regout_hbm.at[pl.ds((b * 4 + 2) * HW + q * PPW, PPW)], sem),
        pltpu.async_copy(
            rb, regout_hbm.at[pl.ds((b * 4 + 3) * HW + q * PPW, PPW)], sem),
    ]
    for h in outs:
      h.wait()

  return sc_kernel


@jax.jit
def kernel(cls_logits, cnt_logits, reg_preds, gt_boxes, classes):
  B, _, H, W = cls_logits.shape
  M = classes.shape[1]
  HW = H * W
  sc_kernel = _build(B, H, W, M)
  cls_flat, cnt_flat, reg_flat = sc_kernel(gt_boxes.astype(jnp.float32),
                                           classes.astype(jnp.int32))
  cls_t = cls_flat.reshape(B, HW, 1)
  cnt_t = cnt_flat.reshape(B, HW, 1)
  reg_t = jnp.transpose(reg_flat.reshape(B, 4, HW), (0, 2, 1))
  return cls_t, cnt_t, reg_t


# speedup vs baseline: 4.5924x; 1.0037x over previous
import functools

import jax
import jax.numpy as jnp
from jax import lax
from jax.experimental import pallas as pl
from jax.experimental.pallas import tpu as pltpu
from jax.experimental.pallas import tpu_sc as plsc

_NC = 2
_NS = 16
_L = 16
_NW = _NC * _NS

_STRIDE = 8
_RADIU = 12.0
_LIMIT_LO = -1.0
_LIMIT_HI = 64.0
_BIG = 99999999.0


@functools.lru_cache(maxsize=None)
def _build(B, H, W, M):
  HW = H * W
  WPB = _NW // B
  PPW = HW // WPB
  CH = PPW // _L
  mesh = plsc.VectorSubcoreMesh(core_axis_name="c", subcore_axis_name="s",
                                num_cores=_NC, num_subcores=_NS)

  @functools.partial(
      pl.kernel,
      out_type=(
          jax.ShapeDtypeStruct((B * HW,), jnp.int32),
          jax.ShapeDtypeStruct((B * HW,), jnp.float32),
          jax.ShapeDtypeStruct((B * 4 * HW,), jnp.float32),
      ),
      mesh=mesh,
      compiler_params=pltpu.CompilerParams(needs_layout_passes=False, use_tc_tiling_on_sc=True),
      scratch_types=[
          pltpu.VMEM((M, 4), jnp.float32),
          pltpu.VMEM((B, M), jnp.int32),
          pltpu.VMEM((PPW,), jnp.float32),
          pltpu.VMEM((PPW,), jnp.float32),
          pltpu.VMEM((PPW,), jnp.float32),
          pltpu.VMEM((PPW,), jnp.float32),
          pltpu.VMEM((PPW,), jnp.int32),
          pltpu.VMEM((PPW,), jnp.float32),
          pltpu.VMEM((PPW,), jnp.float32),
          pltpu.SemaphoreType.DMA,
      ],
  )
  def sc_kernel(gt_hbm, cls_hbm, clsout_hbm, cntout_hbm, regout_hbm,
                boxes_v, classes_v, rl, rt, rr, rb, clsp, cntp, areap, sem):
    wid = lax.axis_index("s") * _NC + lax.axis_index("c")
    b = wid // WPB
    q = wid % WPB

    in1 = pltpu.async_copy(gt_hbm.at[b], boxes_v, sem)
    in2 = pltpu.async_copy(cls_hbm, classes_v, sem)

    neg1 = jnp.full((_L,), -1.0, jnp.float32)
    zero_i = jnp.zeros((_L,), jnp.int32)
    big = jnp.full((_L,), _BIG, jnp.float32)

    def init_body(i, carry):
      for u in range(4):
        sl = pl.ds((i * 4 + u) * _L, _L)
        rl[sl] = neg1
        rt[sl] = neg1
        rr[sl] = neg1
        rb[sl] = neg1
        clsp[sl] = zero_i
        cntp[sl] = neg1
        areap[sl] = big
      return carry

    lax.fori_loop(0, CH // 4, init_body, 0)
    in1.wait()
    in2.wait()

    lane = lax.iota(jnp.int32, _L)
    dxl = lane % 3
    dyl = lane // 3
    lane_ok = lane < 9
    p_base = q * PPW
    col0 = jnp.zeros((_L,), jnp.int32)
    col1 = jnp.full((_L,), 1, jnp.int32)
    col2 = jnp.full((_L,), 2, jnp.int32)
    col3 = jnp.full((_L,), 3, jnp.int32)
    bvec = lax.broadcast(b, (_L,))

    ROWS = PPW // W
    row0 = q * ROWS

    def box_body(m, carry):
      mvec = lax.broadcast(m, (_L,))
      y0 = plsc.load_gather(boxes_v, [mvec, col1])
      y1 = plsc.load_gather(boxes_v, [mvec, col3])
      cy = (y0 + y1) * 0.5
      j0 = ((cy + (1024.0 - 16.0)) * 0.125).astype(jnp.int32) - 127
      j0s = j0[0]

      @pl.when((j0s + 2 >= row0) & (j0s < row0 + ROWS))
      def _process():
        x0 = plsc.load_gather(boxes_v, [mvec, col0])
        x1 = plsc.load_gather(boxes_v, [mvec, col2])
        cm = plsc.load_gather(classes_v, [bvec, mvec])
        cx = (x0 + x1) * 0.5
        area = (x1 - x0) * (y1 - y0)
        i0 = ((cx + (1024.0 - 16.0)) * 0.125).astype(jnp.int32) - 127
        ii = i0 + dxl
        jj = j0 + dyl
        valid = (lane_ok & (ii >= 0) & (ii < W) & (jj >= 0) & (jj < H))
        p_local = jj * W + ii - p_base
        in_r = (p_local >= 0) & (p_local < PPW)
        pc = p_local & (PPW - 1)
        xv = (ii * _STRIDE + _STRIDE // 2).astype(jnp.float32)
        yv = (jj * _STRIDE + _STRIDE // 2).astype(jnp.float32)
        l = xv - x0
        t = yv - y0
        r = x1 - xv
        bb = y1 - yv
        off_min = jnp.minimum(jnp.minimum(l, t), jnp.minimum(r, bb))
        off_max = jnp.maximum(jnp.maximum(l, t), jnp.maximum(r, bb))
        c_off = jnp.maximum(jnp.abs(xv - cx), jnp.abs(yv - cy))
        mask = (valid & in_r & (off_min > 0.0)
                & (off_max > _LIMIT_LO) & (off_max <= _LIMIT_HI)
                & (c_off < _RADIU))
        best = plsc.load_gather(areap, [pc])
        upd = mask & (area < best)
        ratio = (jnp.minimum(l, r) * jnp.minimum(t, bb)) / (
            jnp.maximum(l, r) * jnp.maximum(t, bb) + 1e-10)
        xi = plsc.bitcast(ratio, jnp.int32)
        y = plsc.bitcast(0x5F3759DF - (xi >> 1), jnp.float32)
        y = y * (1.5 - 0.5 * ratio * y * y)
        y = y * (1.5 - 0.5 * ratio * y * y)
        y = y * (1.5 - 0.5 * ratio * y * y)
        cnt = ratio * y
        plsc.store_scatter(areap, [pc], area, mask=upd)
        plsc.store_scatter(cntp, [pc], cnt, mask=upd)
        plsc.store_scatter(clsp, [pc], cm, mask=upd)
        plsc.store_scatter(rl, [pc], l, mask=upd)
        plsc.store_scatter(rt, [pc], t, mask=upd)
        plsc.store_scatter(rr, [pc], r, mask=upd)
        plsc.store_scatter(rb, [pc], bb, mask=upd)

      return carry

    lax.fori_loop(0, M, box_body, 0)

    outs = [
        pltpu.async_copy(clsp, clsout_hbm.at[pl.ds(wid * PPW, PPW)], sem),
        pltpu.async_copy(cntp, cntout_hbm.at[pl.ds(wid * PPW, PPW)], sem),
        pltpu.async_copy(
            rl, regout_hbm.at[pl.ds((b * 4 + 0) * HW + q * PPW, PPW)], sem),
        pltpu.async_copy(
            rt, regout_hbm.at[pl.ds((b * 4 + 1) * HW + q * PPW, PPW)], sem),
        pltpu.async_copy(
            rr, regout_hbm.at[pl.ds((b * 4 + 2) * HW + q * PPW, PPW)], sem),
        pltpu.async_copy(
            rb, regout_hbm.at[pl.ds((b * 4 + 3) * HW + q * PPW, PPW)], sem),
    ]
    for h in outs:
      h.wait()

  return sc_kernel


@jax.jit
def kernel(cls_logits, cnt_logits, reg_preds, gt_boxes, classes):
  B, _, H, W = cls_logits.shape
  M = classes.shape[1]
  HW = H * W
  sc_kernel = _build(B, H, W, M)
  cls_flat, cnt_flat, reg_flat = sc_kernel(gt_boxes.astype(jnp.float32),
                                           classes.astype(jnp.int32))
  cls_t = cls_flat.reshape(B, HW, 1)
  cnt_t = cnt_flat.reshape(B, HW, 1)
  reg_t = jnp.transpose(reg_flat.reshape(B, 4, HW), (0, 2, 1))
  return cls_t, cnt_t, reg_t
